# Initial kernel scaffold; baseline (speedup 1.0000x reference)
import jax
import jax.numpy as jnp
from jax.experimental import pallas as pl


def kernel(rnn_out, char_seq, mask):
    raise NotImplementedError("write your pallas kernel here")


# speedup vs baseline: 2.2920x; 2.2920x over previous
import functools

import jax
import jax.numpy as jnp
from jax import lax
from jax.experimental import pallas as pl
from jax.experimental.pallas import tpu as pltpu
from jax.experimental.pallas import tpu_sc as plsc

B, T, D = 16, 2048, 256
S = 256
WB = 3
L = 16
CHUNK = 32
NCH = S // CHUNK


def _sc_body(rnn_hbm, char_hbm, zrows_hbm, out_hbm, char_v, idx_v, buf_v):
    cid = lax.axis_index("c")
    sid = lax.axis_index("s")
    b = sid
    delta = 1 - 2 * cid

    pltpu.sync_copy(char_hbm.at[b], char_v)

    safe = jnp.full((L,), b * T, dtype=jnp.int32)
    for j in range(S // L):
        idx_v[pl.ds(j * L, L)] = safe

    lane = lax.iota(jnp.int32, L)

    def scan_body(i, cnt):
        c = char_v[pl.ds(i * L, L)]
        p = i * L + lane
        q = p + delta
        m = (c == WB) & (q >= 0) & (q <= T - 1)
        mi = m.astype(jnp.int32)
        slot = plsc.cumsum(mi) - 1 + cnt
        plsc.store_scatter(idx_v, [slot], q + b * T, mask=m)
        return cnt + jnp.sum(mi)

    cnt = lax.fori_loop(0, T // L, scan_body, jnp.int32(0), unroll=False)
    count = jnp.minimum(cnt, S)

    for c in range(NCH):
        @pl.when(c * CHUNK < count)
        def _():
            pltpu.sync_copy(
                rnn_hbm.at[idx_v.at[pl.ds(c * CHUNK, CHUNK)]],
                buf_v.at[pl.ds(c * CHUNK, CHUNK)],
            )

        @pl.when(c * CHUNK >= count)
        def _():
            pltpu.sync_copy(zrows_hbm, buf_v.at[pl.ds(c * CHUNK, CHUNK)])

    zeros = jnp.zeros((L,), dtype=jnp.float32)
    col_vecs = [j * L + lane for j in range(D // L)]
    for r in range(CHUNK):
        rows = jnp.full((L,), r, dtype=jnp.int32) + count
        for cols in col_vecs:
            plsc.store_scatter(buf_v, [rows, cols], zeros)

    pltpu.sync_copy(buf_v.at[pl.ds(0, S)], out_hbm.at[b, :, pl.ds(cid * D, D)])


def kernel(rnn_out, char_seq, mask):
    del mask
    rnn2 = rnn_out.reshape(B * T, D)
    char_seq = char_seq.astype(jnp.int32)
    zrows = jnp.zeros((CHUNK, D), dtype=jnp.float32)

    sc_kernel = pl.kernel(
        _sc_body,
        out_type=jax.ShapeDtypeStruct((B, S, 2 * D), jnp.float32),
        mesh=plsc.VectorSubcoreMesh(
            core_axis_name="c", subcore_axis_name="s",
            num_cores=2, num_subcores=16,
        ),
        scratch_types=[
            pltpu.VMEM((T,), jnp.int32),
            pltpu.VMEM((T,), jnp.int32),
            pltpu.VMEM((S + CHUNK, D), jnp.float32),
        ],
        compiler_params=pltpu.CompilerParams(needs_layout_passes=False),
    )
    return sc_kernel(rnn2, char_seq, zrows)


# speedup vs baseline: 3.0078x; 1.3123x over previous
import jax
import jax.numpy as jnp
from jax import lax
from jax.experimental import pallas as pl
from jax.experimental.pallas import tpu as pltpu
from jax.experimental.pallas import tpu_sc as plsc

B, T, D = 16, 2048, 256
S = 256
WB = 3
L = 16
CHUNK = 32
NCH = S // CHUNK


def _sc_body(rnn_hbm, char_hbm, zrows_hbm, out_hbm,
             char_v, idx_v, buf_v, zrows_v, sem_c, sem_z, sem_g, sem_o):
    cid = lax.axis_index("c")
    sid = lax.axis_index("s")
    b = sid
    delta = 1 - 2 * cid

    char_cp = pltpu.async_copy(char_hbm.at[b], char_v, sem_c)
    z_cp = pltpu.async_copy(zrows_hbm, zrows_v, sem_z)

    safe = jnp.full((L,), b * T, dtype=jnp.int32)
    for j in range(S // L):
        idx_v[pl.ds(j * L, L)] = safe

    char_cp.wait()

    lane = lax.iota(jnp.int32, L)

    def scan_body(i, cnt):
        c = char_v[pl.ds(i * L, L)]
        p = i * L + lane
        q = p + delta
        m = (c == WB) & (q >= 0) & (q <= T - 1)
        mi = m.astype(jnp.int32)
        slot = plsc.cumsum(mi) - 1 + cnt
        plsc.store_scatter(idx_v, [slot], q + b * T, mask=m)
        return cnt + jnp.sum(mi)

    cnt = lax.fori_loop(0, T // L, scan_body, jnp.int32(0), unroll=False)
    count = jnp.minimum(cnt, S)

    z_cp.wait()

    def out_chunk(c):
        return out_hbm.at[b, pl.ds(c * CHUNK, CHUNK), pl.ds(cid * D, D)]

    for c in range(NCH):
        @pl.when(c * CHUNK < count)
        def _():
            pltpu.async_copy(
                rnn_hbm.at[idx_v.at[pl.ds(c * CHUNK, CHUNK)]],
                buf_v.at[pl.ds(c * CHUNK, CHUNK)], sem_g)

        @pl.when(c * CHUNK >= count)
        def _():
            pltpu.async_copy(zrows_v, out_chunk(c), sem_o)

    for c in range(NCH):
        @pl.when(c * CHUNK < count)
        def _():
            pltpu.make_async_copy(
                rnn_hbm.at[idx_v.at[pl.ds(c * CHUNK, CHUNK)]],
                buf_v.at[pl.ds(c * CHUNK, CHUNK)], sem_g).wait()

    zeros = jnp.zeros((L,), dtype=jnp.float32)
    col_vecs = [j * L + lane for j in range(D // L)]
    rem = (CHUNK - count % CHUNK) % CHUNK

    @pl.loop(count, count + rem)
    def _(row):
        rows = jnp.full((L,), 0, dtype=jnp.int32) + row
        for cols in col_vecs:
            plsc.store_scatter(buf_v, [rows, cols], zeros)

    for c in range(NCH):
        @pl.when(c * CHUNK < count)
        def _():
            pltpu.async_copy(buf_v.at[pl.ds(c * CHUNK, CHUNK)],
                             out_chunk(c), sem_o)

    for c in range(NCH):
        pltpu.make_async_copy(zrows_v, out_chunk(c), sem_o).wait()


def kernel(rnn_out, char_seq, mask):
    del mask
    rnn2 = rnn_out.reshape(B * T, D)
    char_seq = char_seq.astype(jnp.int32)
    zrows = jnp.zeros((CHUNK, D), dtype=jnp.float32)

    sc_kernel = pl.kernel(
        _sc_body,
        out_type=jax.ShapeDtypeStruct((B, S, 2 * D), jnp.float32),
        mesh=plsc.VectorSubcoreMesh(
            core_axis_name="c", subcore_axis_name="s",
            num_cores=2, num_subcores=16,
        ),
        scratch_types=[
            pltpu.VMEM((T,), jnp.int32),
            pltpu.VMEM((T,), jnp.int32),
            pltpu.VMEM((S + CHUNK, D), jnp.float32),
            pltpu.VMEM((CHUNK, D), jnp.float32),
            pltpu.SemaphoreType.DMA,
            pltpu.SemaphoreType.DMA,
            pltpu.SemaphoreType.DMA,
            pltpu.SemaphoreType.DMA,
        ],
        compiler_params=pltpu.CompilerParams(needs_layout_passes=False),
    )
    return sc_kernel(rnn2, char_seq, zrows)


# speedup vs baseline: 3.0610x; 1.0177x over previous
import jax
import jax.numpy as jnp
from jax import lax
from jax.experimental import pallas as pl
from jax.experimental.pallas import tpu as pltpu
from jax.experimental.pallas import tpu_sc as plsc

B, T, D = 16, 2048, 256
S = 256
WB = 3
L = 16
CHUNK = 32
NCH = S // CHUNK


def _sc_body(rnn_hbm, char_hbm, out_hbm,
             char_v, idx_v, buf_v, zrows_v, sem_c, sem_g, sem_o):
    cid = lax.axis_index("c")
    sid = lax.axis_index("s")
    b = sid
    delta = 1 - 2 * cid
    lane = lax.iota(jnp.int32, L)
    zeros = jnp.zeros((L,), dtype=jnp.float32)

    char_cp = pltpu.async_copy(char_hbm.at[b], char_v, sem_c)

    safe = jnp.full((L,), b * T, dtype=jnp.int32)
    for j in range(S // L):
        idx_v[pl.ds(j * L, L)] = safe

    @pl.loop(0, CHUNK)
    def _(r):
        for j in range(D // L):
            zrows_v[r, pl.ds(j * L, L)] = zeros

    char_cp.wait()

    def scan_body(i, cnt_v):
        c = char_v[pl.ds(i * L, L)]
        p = i * L + lane
        q = p + delta
        m = (c == WB) & (q >= 0) & (q <= T - 1)
        slot = plsc.cumsum(m.astype(jnp.int32)) - 1 + cnt_v
        plsc.store_scatter(idx_v, [slot], q + b * T, mask=m)
        return cnt_v + plsc.all_reduce_population_count(m)

    cnt_v = lax.fori_loop(0, T // L, scan_body,
                          jnp.zeros((L,), jnp.int32), unroll=4)
    count = jnp.minimum(jnp.max(cnt_v), S)
    nch_g = (count + CHUNK - 1) // CHUNK

    def rows(c):
        return pl.multiple_of(c * CHUNK, CHUNK)

    def out_chunk(c):
        return out_hbm.at[b, pl.ds(rows(c), CHUNK), pl.ds(cid * D, D)]

    @pl.loop(0, nch_g)
    def _(c):
        pltpu.async_copy(rnn_hbm.at[idx_v.at[pl.ds(rows(c), CHUNK)]],
                         buf_v.at[pl.ds(rows(c), CHUNK)], sem_g)

    @pl.loop(nch_g, NCH)
    def _(c):
        pltpu.async_copy(zrows_v, out_chunk(c), sem_o)

    @pl.loop(0, nch_g)
    def _(c):
        pltpu.make_async_copy(rnn_hbm.at[idx_v.at[pl.ds(rows(c), CHUNK)]],
                              buf_v.at[pl.ds(rows(c), CHUNK)], sem_g).wait()

    col_vecs = [j * L + lane for j in range(D // L)]
    rem = (CHUNK - count % CHUNK) % CHUNK

    @pl.loop(count, count + rem)
    def _(row):
        rvec = jnp.zeros((L,), dtype=jnp.int32) + row
        for cols in col_vecs:
            plsc.store_scatter(buf_v, [rvec, cols], zeros)

    @pl.loop(0, nch_g)
    def _(c):
        pltpu.async_copy(buf_v.at[pl.ds(rows(c), CHUNK)], out_chunk(c), sem_o)

    @pl.loop(0, NCH)
    def _(c):
        pltpu.make_async_copy(zrows_v, out_chunk(c), sem_o).wait()


def kernel(rnn_out, char_seq, mask):
    del mask
    rnn2 = rnn_out.reshape(B * T, D)
    char_seq = char_seq.astype(jnp.int32)

    sc_kernel = pl.kernel(
        _sc_body,
        out_type=jax.ShapeDtypeStruct((B, S, 2 * D), jnp.float32),
        mesh=plsc.VectorSubcoreMesh(
            core_axis_name="c", subcore_axis_name="s",
            num_cores=2, num_subcores=16,
        ),
        scratch_types=[
            pltpu.VMEM((T,), jnp.int32),
            pltpu.VMEM((T,), jnp.int32),
            pltpu.VMEM((S + CHUNK, D), jnp.float32),
            pltpu.VMEM((CHUNK, D), jnp.float32),
            pltpu.SemaphoreType.DMA,
            pltpu.SemaphoreType.DMA,
            pltpu.SemaphoreType.DMA,
        ],
        compiler_params=pltpu.CompilerParams(needs_layout_passes=False),
    )
    return sc_kernel(rnn2, char_seq)
